# baseline (speedup 1.0000x reference)
import functools

import jax
import jax.numpy as jnp
from jax import lax
from jax.experimental import pallas as pl
from jax.experimental.pallas import tpu as pltpu
from jax.experimental.pallas import tpu_sc as plsc

EMB = 32
CHUNK = 128
NBUF = 8


@functools.partial(jax.jit, static_argnames=())
def _embed_flat(idx2d, table):
    nrows, chunk = idx2d.shape
    info = plsc.get_sparse_core_info()
    ncores, nsub = info.num_cores, info.num_subcores
    nw = ncores * nsub
    rpw = nrows // nw

    mesh = plsc.VectorSubcoreMesh(core_axis_name="c", subcore_axis_name="s")
    ngroups = rpw // NBUF

    @functools.partial(
        pl.kernel,
        out_type=jax.ShapeDtypeStruct((nrows * chunk, EMB), jnp.float32),
        mesh=mesh,
        scratch_types=(
            [pltpu.VMEM((rpw, chunk), jnp.int32),
             pltpu.VMEM((2, NBUF * chunk, EMB), jnp.float32)]
            + [pltpu.SemaphoreType.DMA] * (2 * NBUF)
        ),
        compiler_params=pltpu.CompilerParams(use_tc_tiling_on_sc=False),
    )
    def body(idx_hbm, table_hbm, out_hbm, idx_v, rows_v, *gsems):
        wid = lax.axis_index("s") * ncores + lax.axis_index("c")
        r0 = wid * rpw
        pltpu.sync_copy(idx_hbm.at[pl.ds(r0, rpw), :], idx_v)

        def fire(g, h):
            for b in range(NBUF):
                pltpu.async_copy(
                    table_hbm.at[idx_v.at[g * NBUF + b]],
                    rows_v.at[h].at[pl.ds(b * chunk, chunk), :],
                    gsems[h * NBUF + b],
                )

        fire(0, 0)

        @pl.loop(0, ngroups)
        def _(g):
            h = lax.rem(g, 2)

            @pl.when(g + 1 < ngroups)
            def _():
                @pl.when(h == 0)
                def _():
                    fire(g + 1, 1)

                @pl.when(h == 1)
                def _():
                    fire(g + 1, 0)

            for hh in range(2):
                @pl.when(h == hh)
                def _():
                    for b in range(NBUF):
                        pltpu.make_async_copy(
                            table_hbm.at[idx_v.at[g * NBUF + b]],
                            rows_v.at[hh].at[pl.ds(b * chunk, chunk), :],
                            gsems[hh * NBUF + b],
                        ).wait()
                    pltpu.sync_copy(
                        rows_v.at[hh],
                        out_hbm.at[
                            pl.ds((r0 + g * NBUF) * chunk, NBUF * chunk), :
                        ],
                    )

    return body(idx2d, table)


def kernel(x_ids, table):
    batch, seq = x_ids.shape
    flat = x_ids.reshape(-1).astype(jnp.int32)
    idx2d = flat.reshape(-1, CHUNK)
    out = _embed_flat(idx2d, table)
    return out.reshape(batch, seq, EMB)


# speedup vs baseline: 1.6080x; 1.6080x over previous
import functools

import jax
import jax.numpy as jnp
from jax import lax
from jax.experimental import pallas as pl
from jax.experimental.pallas import tpu as pltpu
from jax.experimental.pallas import tpu_sc as plsc

EMB = 32
NBUF = 8


def _embed(x_ids, table):
    batch, seq = x_ids.shape
    info = plsc.get_sparse_core_info()
    ncores, nsub = info.num_cores, info.num_subcores
    nw = ncores * nsub
    rpw = batch // nw

    mesh = plsc.VectorSubcoreMesh(core_axis_name="c", subcore_axis_name="s")

    @functools.partial(
        pl.kernel,
        out_type=jax.ShapeDtypeStruct((batch, seq, EMB), jnp.float32),
        mesh=mesh,
        scratch_types=(
            [pltpu.VMEM((rpw, seq), jnp.int32),
             pltpu.VMEM((NBUF, seq, EMB), jnp.float32)]
            + [pltpu.SemaphoreType.DMA] * NBUF
        ),
        compiler_params=pltpu.CompilerParams(use_tc_tiling_on_sc=False),
    )
    def body(idx_hbm, table_hbm, out_hbm, idx_v, rows_v, *gsems):
        wid = lax.axis_index("s") * ncores + lax.axis_index("c")
        r0 = wid * rpw
        pltpu.sync_copy(idx_hbm.at[pl.ds(r0, rpw), :], idx_v)

        for b in range(NBUF):
            pltpu.async_copy(table_hbm.at[idx_v.at[b]], rows_v.at[b], gsems[b])

        @pl.loop(0, rpw, step=NBUF)
        def _(g):
            for b in range(NBUF):
                r = g + b
                pltpu.make_async_copy(
                    table_hbm.at[idx_v.at[r]], rows_v.at[b], gsems[b]
                ).wait()
                pltpu.sync_copy(rows_v.at[b], out_hbm.at[r0 + r])
                nxt = r + NBUF

                @pl.when(nxt < rpw)
                def _():
                    pltpu.async_copy(
                        table_hbm.at[idx_v.at[nxt]], rows_v.at[b], gsems[b]
                    )

    return body(x_ids, table)


def kernel(x_ids, table):
    return _embed(x_ids.astype(jnp.int32), table)


# speedup vs baseline: 1.8785x; 1.1683x over previous
import functools

import jax
import jax.numpy as jnp
from jax import lax
from jax.experimental import pallas as pl
from jax.experimental.pallas import tpu as pltpu
from jax.experimental.pallas import tpu_sc as plsc

EMB = 32
NB = 16
PAD = 17


def _embed_t(x_ids, table):
    batch, seq = x_ids.shape
    info = plsc.get_sparse_core_info()
    ncores, nsub = info.num_cores, info.num_subcores
    nw = ncores * nsub
    rpw = batch // nw
    nblk = rpw // NB
    se = seq * EMB

    mesh = plsc.VectorSubcoreMesh(core_axis_name="c", subcore_axis_name="s")

    @functools.partial(
        pl.kernel,
        out_type=jax.ShapeDtypeStruct((se, batch), jnp.float32),
        mesh=mesh,
        scratch_types=[
            pltpu.VMEM((rpw, seq), jnp.int32),
            pltpu.VMEM((NB * seq, EMB), jnp.float32),
            pltpu.VMEM((NB * seq, EMB), jnp.float32),
            pltpu.VMEM((se, PAD), jnp.float32),
            pltpu.SemaphoreType.DMA,
            pltpu.SemaphoreType.DMA,
        ],
        compiler_params=pltpu.CompilerParams(
            use_tc_tiling_on_sc=False, needs_layout_passes=False
        ),
    )
    def body(idx_hbm, table_hbm, out_hbm, idx_v, rows_a, rows_b, plane,
             sem_a, sem_b):
        wid = lax.axis_index("s") * ncores + lax.axis_index("c")
        r0 = wid * rpw
        pltpu.sync_copy(idx_hbm.at[pl.ds(r0, rpw), :], idx_v)

        def fire(k, rows, sem):
            for b in range(NB):
                pltpu.async_copy(
                    table_hbm.at[idx_v.at[k * NB + b]],
                    rows.at[pl.ds(b * seq, seq), :],
                    sem,
                )

        def drain(k, rows, sem):
            for b in range(NB):
                pltpu.make_async_copy(
                    table_hbm.at[idx_v.at[k * NB + b]],
                    rows.at[pl.ds(b * seq, seq), :],
                    sem,
                ).wait()

        lane = lax.iota(jnp.int32, 16)
        bsplat = [jnp.full((16,), b, jnp.int32) for b in range(NB)]

        def consume(k, rows, sem):
            drain(k, rows, sem)

            @plsc.parallel_loop(0, seq, unroll=2)
            def _(s):
                ids_lo = lane + s * EMB
                ids_hi = ids_lo + 16
                for b in range(NB):
                    j = b * seq + s
                    v0 = rows[j, pl.ds(0, 16)]
                    v1 = rows[j, pl.ds(16, 16)]
                    plsc.store_scatter(plane, [ids_lo, bsplat[b]], v0)
                    plsc.store_scatter(plane, [ids_hi, bsplat[b]], v1)

            pltpu.sync_copy(
                plane.at[:, pl.ds(0, NB)],
                out_hbm.at[:, pl.ds(r0 + k * NB, NB)],
            )

        fire(0, rows_a, sem_a)

        @pl.loop(0, nblk)
        def _(k):
            h = lax.rem(k, 2)

            @pl.when(h == 0)
            def _():
                @pl.when(k + 1 < nblk)
                def _():
                    fire(k + 1, rows_b, sem_b)

                consume(k, rows_a, sem_a)

            @pl.when(h == 1)
            def _():
                @pl.when(k + 1 < nblk)
                def _():
                    fire(k + 1, rows_a, sem_a)

                consume(k, rows_b, sem_b)

    return body(x_ids, table)


def kernel(x_ids, table):
    batch, seq = x_ids.shape
    out = _embed_t(x_ids.astype(jnp.int32), table)
    return jnp.transpose(out.reshape(seq, EMB, batch), (2, 0, 1))
